# Initial kernel scaffold; baseline (speedup 1.0000x reference)
import jax
import jax.numpy as jnp
from jax.experimental import pallas as pl


def kernel(x, W, b_enc, b_dec):
    raise NotImplementedError("write your pallas kernel here")


# speedup vs baseline: 2.4139x; 2.4139x over previous
import functools

import jax
import jax.numpy as jnp
from jax.experimental import pallas as pl
from jax.experimental.pallas import tpu as pltpu

INPUT_DIM = 4096
HIDDEN_DIM = 16384
K = 256
SPARSITY_COEF = 0.001

ENC_TILE = 1024
ENC_NT = HIDDEN_DIM // ENC_TILE
DEC_TILE = 1024
DEC_NT = HIDDEN_DIM // DEC_TILE


def _encode_topk_kernel(x_ref, w_ref, b_ref, lat_ref, enc_scr):
    i = pl.program_id(0)
    acc = jax.lax.dot_general(
        x_ref[...], w_ref[...], (((1,), (1,)), ((), ())),
        preferred_element_type=jnp.float32)
    enc = jnp.clip(acc + b_ref[...], -10.0, 10.0)
    enc_scr[:, pl.ds(i * ENC_TILE, ENC_TILE)] = enc

    @pl.when(i == ENC_NT - 1)
    def _():
        e = enc_scr[...]
        bits = jax.lax.bitcast_convert_type(e, jnp.int32)
        s = jnp.where(bits >= 0, bits, bits ^ jnp.int32(0x7FFFFFFF))
        us = jax.lax.bitcast_convert_type(s, jnp.uint32) ^ jnp.uint32(0x80000000)

        def tbody(b, t):
            cand = t | (jnp.uint32(1) << (31 - b))
            cnt = jnp.sum((us >= cand).astype(jnp.int32), axis=1, keepdims=True)
            return jnp.where(cnt >= K, cand, t)

        t = jax.lax.fori_loop(0, 32, tbody, jnp.zeros((32, 1), jnp.uint32))
        cnt_gt = jnp.sum((us > t).astype(jnp.int32), axis=1, keepdims=True)
        r = K - cnt_gt
        eq = us == t
        iota = jax.lax.broadcasted_iota(jnp.int32, (32, HIDDEN_DIM), 1)

        def jbody(b, J):
            cand = J + (jnp.int32(1) << (14 - b))
            cnt = jnp.sum((eq & (iota < cand)).astype(jnp.int32),
                          axis=1, keepdims=True)
            return jnp.where(cnt < r, cand, J)

        J = jax.lax.fori_loop(0, 15, jbody, jnp.zeros((32, 1), jnp.int32))
        sel = (us > t) | (eq & (iota <= J))
        lat_ref[...] = jnp.where(sel & (e > 0.0), e, 0.0)


def _decode_loss_kernel(lat_ref, w_ref, x_ref, b_ref, rec_ref, sq_ref, ab_ref):
    i = pl.program_id(0)

    @pl.when(i == 0)
    def _():
        rec_ref[...] = jnp.zeros_like(rec_ref)
        ab_ref[...] = jnp.zeros_like(ab_ref)

    lat = lat_ref[...]
    rec_ref[...] += jax.lax.dot_general(
        lat, w_ref[...], (((1,), (0,)), ((), ())),
        preferred_element_type=jnp.float32)
    ab_ref[...] += jnp.sum(lat).reshape(1, 1)

    @pl.when(i == DEC_NT - 1)
    def _():
        rec = rec_ref[...] + b_ref[...]
        rec_ref[...] = rec
        sq_ref[...] = jnp.sum((rec - x_ref[...]) ** 2).reshape(1, 1)


@functools.partial(jax.jit, static_argnames=())
def kernel(x, W, b_enc, b_dec):
    B, T, C = x.shape
    x_flat = x.reshape(B * T, C)

    latent = pl.pallas_call(
        _encode_topk_kernel,
        grid=(ENC_NT,),
        in_specs=[
            pl.BlockSpec((B * T, C), lambda i: (0, 0)),
            pl.BlockSpec((ENC_TILE, C), lambda i: (i, 0)),
            pl.BlockSpec((1, ENC_TILE), lambda i: (0, i)),
        ],
        out_specs=pl.BlockSpec((B * T, HIDDEN_DIM), lambda i: (0, 0)),
        out_shape=jax.ShapeDtypeStruct((B * T, HIDDEN_DIM), jnp.float32),
        scratch_shapes=[pltpu.VMEM((B * T, HIDDEN_DIM), jnp.float32)],
    )(x_flat, W, b_enc.reshape(1, HIDDEN_DIM))

    recon, sq_sum, abs_sum = pl.pallas_call(
        _decode_loss_kernel,
        grid=(DEC_NT,),
        in_specs=[
            pl.BlockSpec((B * T, DEC_TILE), lambda i: (0, i)),
            pl.BlockSpec((DEC_TILE, C), lambda i: (i, 0)),
            pl.BlockSpec((B * T, C), lambda i: (0, 0)),
            pl.BlockSpec((1, C), lambda i: (0, 0)),
        ],
        out_specs=[
            pl.BlockSpec((B * T, C), lambda i: (0, 0)),
            pl.BlockSpec((1, 1), lambda i: (0, 0)),
            pl.BlockSpec((1, 1), lambda i: (0, 0)),
        ],
        out_shape=[
            jax.ShapeDtypeStruct((B * T, C), jnp.float32),
            jax.ShapeDtypeStruct((1, 1), jnp.float32),
            jax.ShapeDtypeStruct((1, 1), jnp.float32),
        ],
    )(latent, W, x_flat, b_dec.reshape(1, C))

    recon_loss = jnp.minimum(sq_sum[0, 0] / (B * T * C), 100.0)
    sparsity_loss = jnp.minimum(abs_sum[0, 0] / (B * T * HIDDEN_DIM), 10.0)
    sae_loss = recon_loss + SPARSITY_COEF * sparsity_loss
    return (recon.reshape(B, T, C), latent.reshape(B, T, HIDDEN_DIM), sae_loss)
